# baseline (speedup 1.0000x reference)
import jax
import jax.numpy as jnp
from jax.experimental import pallas as pl
from jax.experimental.pallas import tpu as pltpu

_TM = 1024


def _body(b_smem, x_ref, w_ref, o_ref, wm_ref, bias_ref):
    i = pl.program_id(0)
    R, E = b_smem.shape

    @pl.when(i == 0)
    def _init():
        wm_ref[...] = (w_ref[0] + w_ref[1] + w_ref[2] + w_ref[3]) * 0.25
        cols = jax.lax.broadcasted_iota(jnp.int32, bias_ref.shape, 1)
        acc = jnp.zeros(bias_ref.shape, jnp.float32)
        for e in range(E):
            s = 0.25 * (
                b_smem[0, e] + b_smem[1, e] + b_smem[2, e] + b_smem[3, e]
            )
            acc = acc + jnp.where(cols == e, s, 0.0)
        bias_ref[...] = acc

    o_ref[pl.ds(i * _TM, _TM), :] = (
        jnp.dot(x_ref[...], wm_ref[...], preferred_element_type=jnp.float32)
        + bias_ref[0, :]
    )


def kernel(x, W, b):
    T, D = x.shape
    R, _, E = W.shape
    grid_spec = pltpu.PrefetchScalarGridSpec(
        num_scalar_prefetch=1,
        grid=(T // _TM,),
        in_specs=[
            pl.BlockSpec((_TM, D), lambda i, b_s: (i, 0)),
            pl.BlockSpec((R, D, E), lambda i, b_s: (0, 0, 0)),
        ],
        out_specs=pl.BlockSpec((T, E), lambda i, b_s: (0, 0)),
        scratch_shapes=[
            pltpu.VMEM((D, E), jnp.float32),
            pltpu.VMEM((8, E), jnp.float32),
        ],
    )
    return pl.pallas_call(
        _body,
        grid_spec=grid_spec,
        out_shape=jax.ShapeDtypeStruct((T, E), jnp.float32),
        compiler_params=pltpu.CompilerParams(
            dimension_semantics=("arbitrary",),
        ),
    )(b, x, W)


# speedup vs baseline: 1.0154x; 1.0154x over previous
import jax
import jax.numpy as jnp
from jax.experimental import pallas as pl
from jax.experimental.pallas import tpu as pltpu

_TM = 1024


def _body(b_smem, x_ref, w_ref, o_ref, wm_ref, bias_ref):
    i = pl.program_id(0)
    R, E = b_smem.shape

    @pl.when(i == 0)
    def _init():
        wm_ref[...] = (w_ref[0] + w_ref[1] + w_ref[2] + w_ref[3]) * 0.25
        cols = jax.lax.broadcasted_iota(jnp.int32, bias_ref.shape, 1)
        acc = jnp.zeros(bias_ref.shape, jnp.float32)
        for e in range(E):
            s = 0.25 * (
                b_smem[0, e] + b_smem[1, e] + b_smem[2, e] + b_smem[3, e]
            )
            acc = acc + jnp.where(cols == e, s, 0.0)
        bias_ref[...] = acc

    o_ref[...] = (
        jnp.dot(x_ref[...], wm_ref[...], preferred_element_type=jnp.float32)
        + bias_ref[0, :]
    )


def kernel(x, W, b):
    T, D = x.shape
    R, _, E = W.shape
    grid_spec = pltpu.PrefetchScalarGridSpec(
        num_scalar_prefetch=1,
        grid=(T // _TM,),
        in_specs=[
            pl.BlockSpec((_TM, D), lambda i, b_s: (i, 0)),
            pl.BlockSpec((R, D, E), lambda i, b_s: (0, 0, 0)),
        ],
        out_specs=pl.BlockSpec((_TM, E), lambda i, b_s: (i, 0)),
        scratch_shapes=[
            pltpu.VMEM((D, E), jnp.float32),
            pltpu.VMEM((8, E), jnp.float32),
        ],
    )
    return pl.pallas_call(
        _body,
        grid_spec=grid_spec,
        out_shape=jax.ShapeDtypeStruct((T, E), jnp.float32),
        compiler_params=pltpu.CompilerParams(
            dimension_semantics=("arbitrary",),
        ),
    )(b, x, W)


# speedup vs baseline: 1.0206x; 1.0052x over previous
import jax
import jax.numpy as jnp
from jax.experimental import pallas as pl
from jax.experimental.pallas import tpu as pltpu

_TM = 1024


def _body(x_ref, w_ref, b_ref, o_ref):
    wm = (w_ref[0] + w_ref[1] + w_ref[2] + w_ref[3]) * 0.25
    bm = (b_ref[0] + b_ref[1] + b_ref[2] + b_ref[3]) * 0.25
    o_ref[...] = (
        jnp.dot(x_ref[...], wm, preferred_element_type=jnp.float32) + bm
    )


def kernel(x, W, b):
    T, D = x.shape
    R, _, E = W.shape
    return pl.pallas_call(
        _body,
        grid=(T // _TM,),
        in_specs=[
            pl.BlockSpec((_TM, D), lambda i: (i, 0)),
            pl.BlockSpec((R, D, E), lambda i: (0, 0, 0)),
            pl.BlockSpec((R, E), lambda i: (0, 0)),
        ],
        out_specs=pl.BlockSpec((_TM, E), lambda i: (i, 0)),
        out_shape=jax.ShapeDtypeStruct((T, E), jnp.float32),
        compiler_params=pltpu.CompilerParams(
            dimension_semantics=("parallel",),
        ),
    )(x, W, b)
